# Initial kernel scaffold; baseline (speedup 1.0000x reference)
import jax
import jax.numpy as jnp
from jax.experimental import pallas as pl


def kernel(x, c_x, senders, receivers, W, b, Wb, centers):
    raise NotImplementedError("write your pallas kernel here")


# speedup vs baseline: 8.0269x; 8.0269x over previous
import functools

import jax
import jax.numpy as jnp
from jax import lax
from jax.experimental import pallas as pl
from jax.experimental.pallas import tpu as pltpu
from jax.experimental.pallas import tpu_sc as plsc

N = 10000
E = 320000
D = 128
HID = 128
NUM_REPS = 512

NC = 2
NS = 16
N_PAD = 10240
E_PAD = 327680
CHUNK = 128
E_ROWS = E_PAD // CHUNK
ROWS_PER_WORKER = E_ROWS // (NC * NS)
ROWS_PER_TILE = E_ROWS // NS
GROUP = 16
NODES_PER_TILE = N_PAD // NS
BLK = 512
GRID = N_PAD // BLK

_SELU_ALPHA = 1.6732632423543772
_SELU_SCALE = 1.0507009873554805
_HIGHEST = jax.lax.Precision.HIGHEST

_MESH = plsc.VectorSubcoreMesh(
    core_axis_name="c", subcore_axis_name="s", num_cores=NC, num_subcores=NS)


@functools.partial(
    pl.kernel,
    out_type=pltpu.HBM((NC, 2, N_PAD), jnp.float32),
    mesh=_MESH,
    scratch_types=[
        pltpu.VMEM((ROWS_PER_WORKER, CHUNK), jnp.int32),
        pltpu.VMEM((ROWS_PER_WORKER, CHUNK), jnp.int32),
        pltpu.VMEM((CHUNK,), jnp.float32),
        pltpu.VMEM_SHARED((N_PAD,), jnp.float32),
        pltpu.VMEM_SHARED((N_PAD,), jnp.float32),
        pltpu.SemaphoreType.DMA,
    ],
)
def _deg_kernel(s2d, r2d, zeros_n, out, idx_s, idx_r, ones_b, hist_s, hist_r,
                sem):
    c = lax.axis_index("c")
    s = lax.axis_index("s")
    w = c * NS + s
    for i in range(CHUNK // 16):
        ones_b[pl.ds(i * 16, 16)] = jnp.ones((16,), jnp.float32)
    zsl = pl.ds(s * NODES_PER_TILE, NODES_PER_TILE)
    pltpu.sync_copy(zeros_n.at[zsl], hist_s.at[zsl])
    pltpu.sync_copy(zeros_n.at[zsl], hist_r.at[zsl])
    row0 = w * ROWS_PER_WORKER
    pltpu.sync_copy(s2d.at[pl.ds(row0, ROWS_PER_WORKER), :], idx_s)
    pltpu.sync_copy(r2d.at[pl.ds(row0, ROWS_PER_WORKER), :], idx_r)
    plsc.subcore_barrier()

    def body(j, carry):
        d1 = pltpu.async_copy(ones_b, hist_s.at[idx_s.at[j]], sem, add=True)
        d2 = pltpu.async_copy(ones_b, hist_r.at[idx_r.at[j]], sem, add=True)
        d1.wait()
        d2.wait()
        return carry

    lax.fori_loop(0, ROWS_PER_WORKER, body, 0)
    plsc.subcore_barrier()
    pltpu.sync_copy(hist_s.at[zsl], out.at[c, 0, zsl])
    pltpu.sync_copy(hist_r.at[zsl], out.at[c, 1, zsl])


@functools.partial(
    pl.kernel,
    out_type=pltpu.HBM((NC, N_PAD, D), jnp.float32),
    mesh=_MESH,
    scratch_types=[
        pltpu.VMEM((GROUP, CHUNK), jnp.int32),
        pltpu.VMEM((GROUP, CHUNK), jnp.int32),
        pltpu.VMEM((2, CHUNK, D), jnp.float32),
        pltpu.VMEM_SHARED((N_PAD, D), jnp.float32),
        pltpu.SemaphoreType.DMA((2,)),
    ],
)
def _agg_kernel(hcat, soff, r2d, zeros2d, out, sidx, ridx, rows, agg, gsem):
    c = lax.axis_index("c")
    s = lax.axis_index("s")
    zsl = pl.ds(s * NODES_PER_TILE, NODES_PER_TILE)
    pltpu.sync_copy(zeros2d.at[zsl, :], agg.at[zsl, :])
    plsc.subcore_barrier()
    row0 = s * ROWS_PER_TILE

    def group(g, carry):
        gr = row0 + g * GROUP
        pltpu.sync_copy(soff.at[c, pl.ds(gr, GROUP), :], sidx)
        pltpu.sync_copy(r2d.at[pl.ds(gr, GROUP), :], ridx)
        pltpu.async_copy(hcat.at[sidx.at[0]], rows.at[0], gsem.at[0])

        def body(j, carry2):
            b = lax.rem(j, 2)
            nb = lax.rem(j + 1, 2)

            @pl.when(j + 1 < GROUP)
            def _():
                pltpu.async_copy(hcat.at[sidx.at[j + 1]], rows.at[nb],
                                 gsem.at[nb])

            pltpu.make_async_copy(hcat.at[sidx.at[j]], rows.at[b],
                                  gsem.at[b]).wait()
            pltpu.sync_copy(rows.at[b], agg.at[ridx.at[j]], add=True)
            return carry2

        lax.fori_loop(0, GROUP, body, 0)
        return carry

    lax.fori_loop(0, ROWS_PER_TILE // GROUP, group, 0)
    plsc.subcore_barrier()
    pltpu.sync_copy(agg.at[zsl, :], out.at[c, zsl, :])


def _mm_scale_body(xc_ref, w_ref, b_ref, degs_ref, out_ref):
    i = pl.program_id(0)
    z = jnp.dot(xc_ref[...], w_ref[...],
                preferred_element_type=jnp.float32) + b_ref[...]
    dsum = degs_ref[...][:, 0:1] + degs_ref[...][:, 1:2]
    ss = lax.rsqrt(jnp.maximum(dsum, 1.0))
    nid = (lax.rem(i, GRID) * BLK
           + lax.broadcasted_iota(jnp.int32, (BLK, 1), 0))
    ss = jnp.where(nid < N, ss, 0.0)
    out_ref[...] = z * ss


def _mm_scale(xc, w, b2, deg_s2):
    return pl.pallas_call(
        _mm_scale_body,
        grid=(2 * GRID,),
        in_specs=[
            pl.BlockSpec((BLK, D), lambda i: (i, 0)),
            pl.BlockSpec((D, HID), lambda i: (0, 0)),
            pl.BlockSpec((1, HID), lambda i: (0, 0)),
            pl.BlockSpec((BLK, 2), lambda i: (lax.rem(i, GRID), 0)),
        ],
        out_specs=pl.BlockSpec((BLK, HID), lambda i: (i, 0)),
        out_shape=jax.ShapeDtypeStruct((2 * N_PAD, HID), jnp.float32),
    )(xc, w, b2, deg_s2)


def _selu(x):
    return _SELU_SCALE * jnp.where(x > 0, x, _SELU_ALPHA * (jnp.exp(x) - 1.0))


def _post1_body(agg1_ref, agg2_ref, degr_ref, n1_ref, n2_ref, cs_ref):
    i = pl.program_id(0)
    rr = lax.rsqrt(jnp.maximum(
        degr_ref[...][:, 0:1] + degr_ref[...][:, 1:2], 1.0))
    n1 = _selu(agg1_ref[...] * rr)
    n2 = _selu(agg2_ref[...] * rr)
    n1_ref[...] = n1
    n2_ref[...] = n2

    @pl.when(i == 0)
    def _():
        cs_ref[...] = jnp.zeros_like(cs_ref)

    cs_ref[...] += jnp.sum(n1, axis=0, keepdims=True)


def _post1(agg1, agg2, deg_r2):
    return pl.pallas_call(
        _post1_body,
        grid=(GRID,),
        in_specs=[
            pl.BlockSpec((BLK, HID), lambda i: (i, 0)),
            pl.BlockSpec((BLK, HID), lambda i: (i, 0)),
            pl.BlockSpec((BLK, 2), lambda i: (i, 0)),
        ],
        out_specs=[
            pl.BlockSpec((BLK, HID), lambda i: (i, 0)),
            pl.BlockSpec((BLK, HID), lambda i: (i, 0)),
            pl.BlockSpec((1, HID), lambda i: (0, 0)),
        ],
        out_shape=[
            jax.ShapeDtypeStruct((N_PAD, HID), jnp.float32),
            jax.ShapeDtypeStruct((N_PAD, HID), jnp.float32),
            jax.ShapeDtypeStruct((1, HID), jnp.float32),
        ],
    )(agg1, agg2, deg_r2)


def _post2_body(n1_ref, n2_ref, cs_ref, wb_ref, cen_ref, ones_ref,
                h_ref, rep_ref, l1_ref, l2_ref, loss_ref):
    i = pl.program_id(0)
    summ = jax.nn.sigmoid(cs_ref[...] * (1.0 / N))
    v = lax.dot_general(summ, wb_ref[...], (((1,), (1,)), ((), ())))
    n1 = n1_ref[...]
    n2 = n2_ref[...]
    l1_ref[...] = lax.dot_general(n1, v, (((1,), (1,)), ((), ())))
    l2_ref[...] = lax.dot_general(n2, v, (((1,), (1,)), ((), ())))
    nrm = jnp.sqrt(jnp.sum(n1 * n1, axis=1, keepdims=True))
    h = n1 / jnp.maximum(nrm, 1e-12)
    h_ref[...] = h
    cen = cen_ref[...]
    hh = jnp.sum(h * h, axis=1, keepdims=True)
    cc = lax.dot_general(ones_ref[...], cen * cen,
                         (((1,), (1,)), ((), ())), precision=_HIGHEST)
    g = lax.dot_general(h, cen, (((1,), (1,)), ((), ())))
    d2 = hh + cc - 2.0 * g
    dists = jnp.sqrt(jnp.maximum(d2, 0.0) + 1e-12)
    mind = jnp.min(dists, axis=1, keepdims=True)
    ids = lax.broadcasted_iota(jnp.int32, (BLK, NUM_REPS), 1)
    rep_ref[...] = jnp.min(jnp.where(dists <= mind, ids, NUM_REPS), axis=1,
                           keepdims=True)
    rid = i * BLK + lax.broadcasted_iota(jnp.int32, (BLK, 1), 0)
    contrib = jnp.sum(jnp.where(rid < N, mind, 0.0), keepdims=True)

    @pl.when(i == 0)
    def _():
        loss_ref[...] = jnp.zeros_like(loss_ref)

    loss_ref[...] += contrib


def _post2(n1, n2, colsum, wb, centers, ones_row):
    return pl.pallas_call(
        _post2_body,
        grid=(GRID,),
        in_specs=[
            pl.BlockSpec((BLK, HID), lambda i: (i, 0)),
            pl.BlockSpec((BLK, HID), lambda i: (i, 0)),
            pl.BlockSpec((1, HID), lambda i: (0, 0)),
            pl.BlockSpec((HID, HID), lambda i: (0, 0)),
            pl.BlockSpec((NUM_REPS, HID), lambda i: (0, 0)),
            pl.BlockSpec((1, HID), lambda i: (0, 0)),
        ],
        out_specs=[
            pl.BlockSpec((BLK, HID), lambda i: (i, 0)),
            pl.BlockSpec((BLK, 1), lambda i: (i, 0)),
            pl.BlockSpec((BLK, 1), lambda i: (i, 0)),
            pl.BlockSpec((BLK, 1), lambda i: (i, 0)),
            pl.BlockSpec((1, 1), lambda i: (0, 0)),
        ],
        out_shape=[
            jax.ShapeDtypeStruct((N_PAD, HID), jnp.float32),
            jax.ShapeDtypeStruct((N_PAD, 1), jnp.int32),
            jax.ShapeDtypeStruct((N_PAD, 1), jnp.float32),
            jax.ShapeDtypeStruct((N_PAD, 1), jnp.float32),
            jax.ShapeDtypeStruct((1, 1), jnp.float32),
        ],
    )(n1, n2, colsum, wb, centers, ones_row)


def kernel(x, c_x, senders, receivers, W, b, Wb, centers):
    npad = E_PAD - E
    pad_idx = (N + jnp.arange(npad, dtype=jnp.int32) % (N_PAD - N))
    s_pad = jnp.concatenate([senders, pad_idx])
    r_pad = jnp.concatenate([receivers, pad_idx])
    s2d = s_pad.reshape(E_ROWS, CHUNK)
    r2d = r_pad.reshape(E_ROWS, CHUNK)
    soff = jnp.stack([s2d, s2d + N_PAD])

    zeros_n = jnp.zeros((N_PAD,), jnp.float32)
    zeros2d = jnp.zeros((N_PAD, D), jnp.float32)

    deg = _deg_kernel(s2d, r2d, zeros_n)
    deg_s2 = deg[:, 0, :].T
    deg_r2 = deg[:, 1, :].T

    xc = jnp.concatenate([
        jnp.pad(x, ((0, N_PAD - N), (0, 0))),
        jnp.pad(c_x, ((0, N_PAD - N), (0, 0))),
    ])
    hcat = _mm_scale(xc, W, b.reshape(1, HID), deg_s2)

    agg = _agg_kernel(hcat, soff, r2d, zeros2d)

    n1, n2, colsum = _post1(agg[0], agg[1], deg_r2)
    ones_row = jnp.ones((1, HID), jnp.float32)
    h_full, rep, l1, l2, loss = _post2(n1, n2, colsum, Wb, centers, ones_row)

    h = h_full[:N]
    rep_ids = rep[:N, 0]
    logits = jnp.concatenate([l1[:N, 0], l2[:N, 0]])
    cluster_loss = loss[0, 0]
    return (h, centers, rep_ids, cluster_loss, logits)
